# baseline (speedup 1.0000x reference)
import functools

import jax
import jax.numpy as jnp
from jax import lax
from jax.experimental import pallas as pl
from jax.experimental.pallas import tpu as pltpu
from jax.experimental.pallas import tpu_sc as plsc

_JITTER = 0.01
_EPAD = 128
_BM = 128
_BF = 512
_BH = _BF // 2
_NW = 32
_NC = 2
_L = 16


def _routing_body(x_ref, gw_ref, logits_ref, ipack_ref, wpack_ref,
                  counts_ref, carry_ref, *, n_experts, nsteps):
    i = pl.program_id(0)

    @pl.when(i == 0)
    def _():
        carry_ref[...] = jnp.zeros_like(carry_ref)

    x = x_ref[...]
    logits = lax.dot_general(x, gw_ref[...], (((1,), (1,)), ((), ())),
                             preferred_element_type=jnp.float32)
    logits_ref[...] = logits
    bm = logits.shape[0]
    col = lax.broadcasted_iota(jnp.int32, logits.shape, 1)
    valid = col < n_experts
    neg_inf = jnp.float32(-jnp.inf)
    big = jnp.int32(_EPAD)
    scores = jnp.where(valid, logits, neg_inf)

    m1 = jnp.max(scores, axis=1, keepdims=True)
    i1 = jnp.min(jnp.where(scores == m1, col, big), axis=1, keepdims=True)
    sel1 = col == i1
    factor1 = jnp.maximum(jnp.abs(scores), m1)
    mask1 = (m1 - scores) / factor1 > 2.0 * _JITTER
    e1 = jnp.exp(jnp.where(mask1, neg_inf, scores) - m1)
    p1 = e1 / jnp.sum(e1, axis=1, keepdims=True)
    mult1 = jnp.sum(jnp.where(sel1, p1, 0.0), axis=1, keepdims=True)

    scores2 = jnp.where(sel1, neg_inf, scores)
    m2 = jnp.max(scores2, axis=1, keepdims=True)
    i2 = jnp.min(jnp.where(scores2 == m2, col, big), axis=1, keepdims=True)
    sel2 = col == i2
    factor2 = jnp.maximum(jnp.abs(scores), m2)
    mask2 = (m2 - scores) / factor2 > 2.0 * _JITTER
    e2 = jnp.exp(jnp.where(mask2, neg_inf, scores2) - m2)
    p2 = e2 / jnp.sum(e2, axis=1, keepdims=True)
    mult2 = jnp.sum(jnp.where(sel2, p2, 0.0), axis=1, keepdims=True)

    c = sel1.astype(jnp.float32) + sel2.astype(jnp.float32)
    ri = lax.broadcasted_iota(jnp.int32, (bm, bm), 0)
    ci = lax.broadcasted_iota(jnp.int32, (bm, bm), 1)
    tri = (ci < ri).astype(jnp.float32)
    pref = lax.dot_general(tri, c, (((1,), (0,)), ((), ())),
                           preferred_element_type=jnp.float32)
    pref = pref + carry_ref[...]
    rank1 = jnp.sum(jnp.where(sel1, pref, 0.0), axis=1, keepdims=True)
    rank2 = jnp.sum(jnp.where(sel2, pref, 0.0), axis=1, keepdims=True)
    carry_ref[...] = carry_ref[...] + jnp.sum(c, axis=0, keepdims=True)

    @pl.when(i == nsteps - 1)
    def _():
        counts_ref[...] = carry_ref[...]

    zi = jnp.int32(0)
    ipack_ref[...] = (jnp.where(col == 0, i1, zi)
                      + jnp.where(col == 1, i2, zi)
                      + jnp.where(col == 2, rank1.astype(jnp.int32), zi)
                      + jnp.where(col == 3, rank2.astype(jnp.int32), zi))
    wpack_ref[...] = (jnp.where(col < _L, mult1, 0.0)
                      + jnp.where(jnp.logical_and(col >= _L, col < 2 * _L),
                                  mult2, 0.0))


def _routing_call(x, gw_pad, n_experts):
    t, h = x.shape
    bm = min(256, t)
    nsteps = t // bm
    return pl.pallas_call(
        functools.partial(_routing_body, n_experts=n_experts, nsteps=nsteps),
        grid=(nsteps,),
        in_specs=[
            pl.BlockSpec((bm, h), lambda i: (i, 0)),
            pl.BlockSpec((_EPAD, h), lambda i: (0, 0)),
        ],
        out_specs=[
            pl.BlockSpec((bm, _EPAD), lambda i: (i, 0)),
            pl.BlockSpec((bm, _EPAD), lambda i: (i, 0)),
            pl.BlockSpec((bm, _EPAD), lambda i: (i, 0)),
            pl.BlockSpec((1, _EPAD), lambda i: (0, 0)),
        ],
        out_shape=[
            jax.ShapeDtypeStruct((t, _EPAD), jnp.float32),
            jax.ShapeDtypeStruct((t, _EPAD), jnp.int32),
            jax.ShapeDtypeStruct((t, _EPAD), jnp.float32),
            jax.ShapeDtypeStruct((1, _EPAD), jnp.float32),
        ],
        scratch_shapes=[pltpu.VMEM((1, _EPAD), jnp.float32)],
    )(x, gw_pad)


def _gmm_body(gid_ref, nu_ref, ord_ref, rg_ref, rpp_ref, xs_ref, w1_ref,
              w3_ref, w2_ref, out_ref, w1b, w3b, w2b, sems, *, nf_steps):
    nf = pl.program_id(0)
    l = pl.program_id(1)
    rpp = rpp_ref[0]
    nu = nu_ref[0]
    j = ord_ref[l]
    r = nf * rpp + j
    slot = lax.rem(r, 3)
    nruns = rpp * nf_steps

    def issue(t):
        sl = lax.rem(t, 3)
        nfn = lax.div(t, rpp)
        g = rg_ref[lax.rem(t, rpp)]
        pltpu.make_async_copy(
            w1_ref.at[g, pl.ds(nfn * _BF, _BF), :], w1b.at[sl],
            sems.at[0, sl]).start()
        pltpu.make_async_copy(
            w3_ref.at[g, pl.ds(nfn * _BF, _BF), :], w3b.at[sl],
            sems.at[1, sl]).start()
        pltpu.make_async_copy(
            w2_ref.at[g, :, pl.ds(nfn * _BF, _BF)], w2b.at[sl],
            sems.at[2, sl]).start()

    first = jnp.logical_and(
        l < nu,
        jnp.logical_or(l == 0,
                       gid_ref[l] != gid_ref[jnp.maximum(l - 1, 0)]))

    @pl.when(jnp.logical_and(nf == 0, l == 0))
    def _():
        issue(jnp.int32(0))

        @pl.when(nruns > 1)
        def _():
            issue(jnp.int32(1))

    @pl.when(first)
    def _():
        @pl.when(r + 2 < nruns)
        def _():
            issue(r + 2)

        pltpu.make_async_copy(
            w1_ref.at[0, pl.ds(0, _BF), :], w1b.at[slot],
            sems.at[0, slot]).wait()
        pltpu.make_async_copy(
            w3_ref.at[0, pl.ds(0, _BF), :], w3b.at[slot],
            sems.at[1, slot]).wait()
        pltpu.make_async_copy(
            w2_ref.at[0, :, pl.ds(0, _BF)], w2b.at[slot],
            sems.at[2, slot]).wait()

    @pl.when(l < nu)
    def _():
        rows = pl.ds(l * _BM, _BM)
        xs = xs_ref[rows, :]
        dims = (((1,), (1,)), ((), ()))
        gv = lax.dot_general(xs, w1b[slot], dims,
                             preferred_element_type=jnp.float32)
        uv = lax.dot_general(xs, w3b[slot], dims,
                             preferred_element_type=jnp.float32)
        a = (gv * lax.logistic(gv)) * uv
        partial = lax.dot_general(a, w2b[slot], dims,
                                  preferred_element_type=jnp.float32)

        @pl.when(nf == 0)
        def _():
            out_ref[rows, :] = partial

        @pl.when(nf != 0)
        def _():
            out_ref[rows, :] = out_ref[rows, :] + partial


def _gmm_call(xs, w1, w3, w2, gid, nu, ordl, rg, rppa):
    rrows, h = xs.shape
    n_experts, ffn, _ = w1.shape
    nf_steps = ffn // _BF
    nl = rrows // _BM
    grid_spec = pltpu.PrefetchScalarGridSpec(
        num_scalar_prefetch=5,
        grid=(nf_steps, nl),
        in_specs=[
            pl.BlockSpec((rrows, h), lambda nf, l, *_: (0, 0)),
            pl.BlockSpec(memory_space=pl.ANY),
            pl.BlockSpec(memory_space=pl.ANY),
            pl.BlockSpec(memory_space=pl.ANY),
        ],
        out_specs=pl.BlockSpec((rrows, h), lambda nf, l, *_: (0, 0)),
        scratch_shapes=[
            pltpu.VMEM((3, _BF, h), jnp.float32),
            pltpu.VMEM((3, _BF, h), jnp.float32),
            pltpu.VMEM((3, h, _BF), jnp.float32),
            pltpu.SemaphoreType.DMA((3, 3)),
        ],
    )
    return pl.pallas_call(
        functools.partial(_gmm_body, nf_steps=nf_steps),
        grid_spec=grid_spec,
        out_shape=jax.ShapeDtypeStruct((rrows, h), jnp.float32),
        compiler_params=pltpu.CompilerParams(
            dimension_semantics=("arbitrary", "arbitrary")),
    )(gid, nu, ordl, rg, rppa, xs, w1, w3, w2)


def _sc_mesh():
    return plsc.VectorSubcoreMesh(core_axis_name="c", subcore_axis_name="s")


def _wid():
    return lax.axis_index("s") * _NC + lax.axis_index("c")


def _positions(iv, rv, rbw_ref, sl, n_experts):
    ic = iv[sl]
    p = rv[sl]
    for e in range(n_experts):
        p = p + jnp.where(ic == e, rbw_ref[pl.ds(e * _L, _L)],
                          jnp.zeros((_L,), jnp.int32))
    return p


def _scatter_call(x, i1, i2, r1, r2, rbw, rrows, n_experts):
    t, h = x.shape
    tpw = t // _NW

    @functools.partial(
        pl.kernel,
        out_type=jax.ShapeDtypeStruct((rrows, h), jnp.float32),
        mesh=_sc_mesh(),
        scratch_types=[
            pltpu.VMEM((tpw,), jnp.int32),
            pltpu.VMEM((tpw,), jnp.int32),
            pltpu.VMEM((tpw,), jnp.int32),
            pltpu.VMEM((tpw,), jnp.int32),
            pltpu.VMEM((_EPAD,), jnp.int32),
            pltpu.VMEM((tpw, h), jnp.float32),
            pltpu.SemaphoreType.DMA,
        ],
    )
    def k(xh, i1h, i2h, r1h, r2h, rbwh, xsh,
          i1v, i2v, r1v, r2v, rbwv, rowsv, sem):
        base = _wid() * tpw
        pltpu.sync_copy(xh.at[pl.ds(base, tpw)], rowsv)
        pltpu.sync_copy(i1h.at[pl.ds(base, tpw)], i1v)
        pltpu.sync_copy(i2h.at[pl.ds(base, tpw)], i2v)
        pltpu.sync_copy(r1h.at[pl.ds(base, tpw)], r1v)
        pltpu.sync_copy(r2h.at[pl.ds(base, tpw)], r2v)
        pltpu.sync_copy(rbwh, rbwv)
        for c in range(tpw // _L):
            sl = pl.ds(c * _L, _L)
            p0 = _positions(i1v, r1v, rbwv, sl, n_experts)
            pltpu.async_copy(rowsv.at[sl], xsh.at[p0], sem).wait()
            p1 = _positions(i2v, r2v, rbwv, sl, n_experts)
            pltpu.async_copy(rowsv.at[sl], xsh.at[p1], sem).wait()

    return k(x, i1, i2, r1, r2, rbw)


def _combine_call(ys, i1, i2, r1, r2, rbw, wwide, t, n_experts):
    rrows, h = ys.shape
    tpw = t // _NW
    hc = h // _L

    @functools.partial(
        pl.kernel,
        out_type=jax.ShapeDtypeStruct((t, h), jnp.float32),
        mesh=_sc_mesh(),
        scratch_types=[
            pltpu.VMEM((tpw,), jnp.int32),
            pltpu.VMEM((tpw,), jnp.int32),
            pltpu.VMEM((tpw,), jnp.int32),
            pltpu.VMEM((tpw,), jnp.int32),
            pltpu.VMEM((_EPAD,), jnp.int32),
            pltpu.VMEM((tpw, _EPAD), jnp.float32),
            pltpu.VMEM((_L, h), jnp.float32),
            pltpu.VMEM((_L, h), jnp.float32),
            pltpu.SemaphoreType.DMA,
        ],
    )
    def k(ysh, i1h, i2h, r1h, r2h, rbwh, wwh, outh,
          i1v, i2v, r1v, r2v, rbwv, wv, b0, b1, sem):
        base = _wid() * tpw
        pltpu.sync_copy(i1h.at[pl.ds(base, tpw)], i1v)
        pltpu.sync_copy(i2h.at[pl.ds(base, tpw)], i2v)
        pltpu.sync_copy(r1h.at[pl.ds(base, tpw)], r1v)
        pltpu.sync_copy(r2h.at[pl.ds(base, tpw)], r2v)
        pltpu.sync_copy(rbwh, rbwv)
        pltpu.sync_copy(wwh.at[pl.ds(base, tpw)], wv)
        for c in range(tpw // _L):
            sl = pl.ds(c * _L, _L)
            p0 = _positions(i1v, r1v, rbwv, sl, n_experts)
            p1 = _positions(i2v, r2v, rbwv, sl, n_experts)
            cp0 = pltpu.async_copy(ysh.at[p0], b0, sem)
            cp1 = pltpu.async_copy(ysh.at[p1], b1, sem)
            cp0.wait()
            cp1.wait()

            def row_body(r, _):
                m1v = wv[c * _L + r, pl.ds(0, _L)]
                m2v = wv[c * _L + r, pl.ds(_L, _L)]

                def col_body(j, _):
                    sj = pl.ds(j * _L, _L)
                    b0[r, sj] = m1v * b0[r, sj] + m2v * b1[r, sj]
                    return 0

                lax.fori_loop(0, hc, col_body, 0)
                return 0

            lax.fori_loop(0, _L, row_body, 0)
            pltpu.sync_copy(b0, outh.at[pl.ds(base + c * _L, _L)])

    return k(ys, i1, i2, r1, r2, rbw, wwide)


def kernel(hidden_states, gate_w, w1, w2, w3):
    b, s, h = hidden_states.shape
    t = b * s
    n_experts, ffn, _ = w1.shape
    top_k = 2
    x = hidden_states.reshape(t, h)
    nl = (t * top_k) // _BM + n_experts
    rrows = nl * _BM

    gw_pad = jnp.zeros((_EPAD, h), jnp.float32).at[:n_experts].set(gate_w)
    logits_pad, ipack, wwide, countsf = _routing_call(x, gw_pad, n_experts)
    router_logits = logits_pad[:, :n_experts]

    i1 = ipack[:, 0]
    i2 = ipack[:, 1]
    r1 = ipack[:, 2]
    r2 = ipack[:, 3]

    counts = countsf[0, :n_experts].astype(jnp.int32)
    tiles_e = (counts + _BM - 1) // _BM
    start = jnp.concatenate(
        [jnp.zeros((1,), jnp.int32), jnp.cumsum(tiles_e)])
    row_base = _BM * start[:n_experts]
    rbw = jnp.repeat(row_base, _L, total_repeat_length=n_experts * _L)
    rbw = jnp.zeros((_EPAD,), jnp.int32).at[:n_experts * _L].set(rbw)
    larange = jnp.arange(nl, dtype=jnp.int32)
    gidraw = jnp.minimum(
        jnp.sum((larange[:, None] >= start[None, 1:]).astype(jnp.int32),
                axis=1), n_experts - 1).astype(jnp.int32)
    nused = start[n_experts]
    lastg = jnp.max(jnp.where(counts > 0,
                              jnp.arange(n_experts, dtype=jnp.int32), 0))
    gid = jnp.where(larange < nused, gidraw, lastg)
    nu = nused[None]
    b_l = jnp.concatenate([jnp.ones((1,), jnp.int32),
                           (gid[1:] != gid[:-1]).astype(jnp.int32)])
    b_l = b_l * (larange < nused).astype(jnp.int32)
    ordl = jnp.cumsum(b_l) - 1
    rppa = jnp.sum(b_l)[None]
    rg = jnp.zeros((n_experts,), jnp.int32).at[
        jnp.where(b_l == 1, ordl, n_experts)].set(gid, mode='drop')

    xs = _scatter_call(x, i1, i2, r1, r2, rbw, rrows, n_experts)
    ys = _gmm_call(xs, w1, w3, w2, gid, nu, ordl, rg, rppa)
    out = _combine_call(ys, i1, i2, r1, r2, rbw, wwide, t, n_experts)
    return out.reshape(b, s, h), router_logits


# speedup vs baseline: 1.0543x; 1.0543x over previous
import functools

import jax
import jax.numpy as jnp
from jax import lax
from jax.experimental import pallas as pl
from jax.experimental.pallas import tpu as pltpu
from jax.experimental.pallas import tpu_sc as plsc

_JITTER = 0.01
_EPAD = 128
_BM = 128
_BF = 512
_BH = _BF // 2
_NW = 32
_NC = 2
_L = 16


def _routing_body(x_ref, gw_ref, logits_ref, ipack_ref, wpack_ref,
                  counts_ref, carry_ref, *, n_experts, nsteps):
    i = pl.program_id(0)

    @pl.when(i == 0)
    def _():
        carry_ref[...] = jnp.zeros_like(carry_ref)

    x = x_ref[...]
    logits = lax.dot_general(x, gw_ref[...], (((1,), (1,)), ((), ())),
                             preferred_element_type=jnp.float32)
    logits_ref[...] = logits
    bm = logits.shape[0]
    col = lax.broadcasted_iota(jnp.int32, logits.shape, 1)
    valid = col < n_experts
    neg_inf = jnp.float32(-jnp.inf)
    big = jnp.int32(_EPAD)
    scores = jnp.where(valid, logits, neg_inf)

    m1 = jnp.max(scores, axis=1, keepdims=True)
    i1 = jnp.min(jnp.where(scores == m1, col, big), axis=1, keepdims=True)
    sel1 = col == i1
    factor1 = jnp.maximum(jnp.abs(scores), m1)
    mask1 = (m1 - scores) / factor1 > 2.0 * _JITTER
    e1 = jnp.exp(jnp.where(mask1, neg_inf, scores) - m1)
    p1 = e1 / jnp.sum(e1, axis=1, keepdims=True)
    mult1 = jnp.sum(jnp.where(sel1, p1, 0.0), axis=1, keepdims=True)

    scores2 = jnp.where(sel1, neg_inf, scores)
    m2 = jnp.max(scores2, axis=1, keepdims=True)
    i2 = jnp.min(jnp.where(scores2 == m2, col, big), axis=1, keepdims=True)
    sel2 = col == i2
    factor2 = jnp.maximum(jnp.abs(scores), m2)
    mask2 = (m2 - scores) / factor2 > 2.0 * _JITTER
    e2 = jnp.exp(jnp.where(mask2, neg_inf, scores2) - m2)
    p2 = e2 / jnp.sum(e2, axis=1, keepdims=True)
    mult2 = jnp.sum(jnp.where(sel2, p2, 0.0), axis=1, keepdims=True)

    c = sel1.astype(jnp.float32) + sel2.astype(jnp.float32)
    ri = lax.broadcasted_iota(jnp.int32, (bm, bm), 0)
    ci = lax.broadcasted_iota(jnp.int32, (bm, bm), 1)
    tri = (ci < ri).astype(jnp.float32)
    pref = lax.dot_general(tri, c, (((1,), (0,)), ((), ())),
                           preferred_element_type=jnp.float32)
    pref = pref + carry_ref[...]
    rank1 = jnp.sum(jnp.where(sel1, pref, 0.0), axis=1, keepdims=True)
    rank2 = jnp.sum(jnp.where(sel2, pref, 0.0), axis=1, keepdims=True)
    carry_ref[...] = carry_ref[...] + jnp.sum(c, axis=0, keepdims=True)

    @pl.when(i == nsteps - 1)
    def _():
        counts_ref[...] = carry_ref[...]

    zi = jnp.int32(0)
    ipack_ref[...] = (jnp.where(col == 0, i1, zi)
                      + jnp.where(col == 1, i2, zi)
                      + jnp.where(col == 2, rank1.astype(jnp.int32), zi)
                      + jnp.where(col == 3, rank2.astype(jnp.int32), zi))
    wpack_ref[...] = (jnp.where(col < _L, mult1, 0.0)
                      + jnp.where(jnp.logical_and(col >= _L, col < 2 * _L),
                                  mult2, 0.0))


def _routing_call(x, gw_pad, n_experts):
    t, h = x.shape
    bm = min(256, t)
    nsteps = t // bm
    return pl.pallas_call(
        functools.partial(_routing_body, n_experts=n_experts, nsteps=nsteps),
        grid=(nsteps,),
        in_specs=[
            pl.BlockSpec((bm, h), lambda i: (i, 0)),
            pl.BlockSpec((_EPAD, h), lambda i: (0, 0)),
        ],
        out_specs=[
            pl.BlockSpec((bm, _EPAD), lambda i: (i, 0)),
            pl.BlockSpec((bm, _EPAD), lambda i: (i, 0)),
            pl.BlockSpec((bm, _EPAD), lambda i: (i, 0)),
            pl.BlockSpec((1, _EPAD), lambda i: (0, 0)),
        ],
        out_shape=[
            jax.ShapeDtypeStruct((t, _EPAD), jnp.float32),
            jax.ShapeDtypeStruct((t, _EPAD), jnp.int32),
            jax.ShapeDtypeStruct((t, _EPAD), jnp.float32),
            jax.ShapeDtypeStruct((1, _EPAD), jnp.float32),
        ],
        scratch_shapes=[pltpu.VMEM((1, _EPAD), jnp.float32)],
    )(x, gw_pad)


def _gmm_body(gid_ref, nu_ref, ord_ref, rg_ref, rpp_ref, xs_ref, w1_ref,
              w3_ref, w2_ref, out_ref, w1b, w3b, w2b, sems, *, nf_steps):
    nf = pl.program_id(0)
    l = pl.program_id(1)
    rpp = rpp_ref[0]
    nu = nu_ref[0]
    j = ord_ref[l]
    r = nf * rpp + j
    slot = lax.rem(r, 2)
    nruns = rpp * nf_steps

    def issue(t):
        sl = lax.rem(t, 2)
        nfn = lax.div(t, rpp)
        g = rg_ref[lax.rem(t, rpp)]
        pltpu.make_async_copy(
            w1_ref.at[g, pl.ds(nfn * _BF, _BF), :], w1b.at[sl],
            sems.at[0, sl]).start()
        pltpu.make_async_copy(
            w3_ref.at[g, pl.ds(nfn * _BF, _BF), :], w3b.at[sl],
            sems.at[1, sl]).start()
        pltpu.make_async_copy(
            w2_ref.at[g, :, pl.ds(nfn * _BF, _BF)], w2b.at[sl],
            sems.at[2, sl]).start()

    first = jnp.logical_and(
        l < nu,
        jnp.logical_or(l == 0,
                       gid_ref[l] != gid_ref[jnp.maximum(l - 1, 0)]))

    @pl.when(jnp.logical_and(nf == 0, l == 0))
    def _():
        issue(jnp.int32(0))

    @pl.when(first)
    def _():
        @pl.when(r + 1 < nruns)
        def _():
            issue(r + 1)

        pltpu.make_async_copy(
            w1_ref.at[0, pl.ds(0, _BF), :], w1b.at[slot],
            sems.at[0, slot]).wait()
        pltpu.make_async_copy(
            w3_ref.at[0, pl.ds(0, _BF), :], w3b.at[slot],
            sems.at[1, slot]).wait()
        pltpu.make_async_copy(
            w2_ref.at[0, :, pl.ds(0, _BF)], w2b.at[slot],
            sems.at[2, slot]).wait()

    @pl.when(l < nu)
    def _():
        rows = pl.ds(l * _BM, _BM)
        xs = xs_ref[rows, :]
        dims = (((1,), (1,)), ((), ()))
        gv = lax.dot_general(xs, w1b[slot], dims,
                             preferred_element_type=jnp.float32)
        uv = lax.dot_general(xs, w3b[slot], dims,
                             preferred_element_type=jnp.float32)
        a = (gv * lax.logistic(gv)) * uv
        partial = lax.dot_general(a, w2b[slot], dims,
                                  preferred_element_type=jnp.float32)

        @pl.when(nf == 0)
        def _():
            out_ref[rows, :] = partial

        @pl.when(nf != 0)
        def _():
            out_ref[rows, :] = out_ref[rows, :] + partial


def _gmm_call(xs, w1, w3, w2, gid, nu, ordl, rg, rppa):
    rrows, h = xs.shape
    n_experts, ffn, _ = w1.shape
    nf_steps = ffn // _BF
    nl = rrows // _BM
    grid_spec = pltpu.PrefetchScalarGridSpec(
        num_scalar_prefetch=5,
        grid=(nf_steps, nl),
        in_specs=[
            pl.BlockSpec((rrows, h), lambda nf, l, *_: (0, 0)),
            pl.BlockSpec(memory_space=pl.ANY),
            pl.BlockSpec(memory_space=pl.ANY),
            pl.BlockSpec(memory_space=pl.ANY),
        ],
        out_specs=pl.BlockSpec((rrows, h), lambda nf, l, *_: (0, 0)),
        scratch_shapes=[
            pltpu.VMEM((2, _BF, h), jnp.float32),
            pltpu.VMEM((2, _BF, h), jnp.float32),
            pltpu.VMEM((2, h, _BF), jnp.float32),
            pltpu.SemaphoreType.DMA((3, 2)),
        ],
    )
    return pl.pallas_call(
        functools.partial(_gmm_body, nf_steps=nf_steps),
        grid_spec=grid_spec,
        out_shape=jax.ShapeDtypeStruct((rrows, h), jnp.float32),
        compiler_params=pltpu.CompilerParams(
            dimension_semantics=("arbitrary", "arbitrary")),
    )(gid, nu, ordl, rg, rppa, xs, w1, w3, w2)


def _sc_mesh():
    return plsc.VectorSubcoreMesh(core_axis_name="c", subcore_axis_name="s")


def _wid():
    return lax.axis_index("s") * _NC + lax.axis_index("c")


def _positions(iv, rv, rbw_ref, sl, n_experts):
    ic = iv[sl]
    p = rv[sl]
    for e in range(n_experts):
        p = p + jnp.where(ic == e, rbw_ref[pl.ds(e * _L, _L)],
                          jnp.zeros((_L,), jnp.int32))
    return p


def _scatter_call(x, i1, i2, r1, r2, rbw, rrows, n_experts):
    t, h = x.shape
    tpw = t // _NW

    @functools.partial(
        pl.kernel,
        out_type=jax.ShapeDtypeStruct((rrows, h), jnp.float32),
        mesh=_sc_mesh(),
        scratch_types=[
            pltpu.VMEM((tpw,), jnp.int32),
            pltpu.VMEM((tpw,), jnp.int32),
            pltpu.VMEM((tpw,), jnp.int32),
            pltpu.VMEM((tpw,), jnp.int32),
            pltpu.VMEM((_EPAD,), jnp.int32),
            pltpu.VMEM((tpw, h), jnp.float32),
            pltpu.SemaphoreType.DMA,
        ],
    )
    def k(xh, i1h, i2h, r1h, r2h, rbwh, xsh,
          i1v, i2v, r1v, r2v, rbwv, rowsv, sem):
        base = _wid() * tpw
        pltpu.sync_copy(xh.at[pl.ds(base, tpw)], rowsv)
        pltpu.sync_copy(i1h.at[pl.ds(base, tpw)], i1v)
        pltpu.sync_copy(i2h.at[pl.ds(base, tpw)], i2v)
        pltpu.sync_copy(r1h.at[pl.ds(base, tpw)], r1v)
        pltpu.sync_copy(r2h.at[pl.ds(base, tpw)], r2v)
        pltpu.sync_copy(rbwh, rbwv)
        for c in range(tpw // _L):
            sl = pl.ds(c * _L, _L)
            p0 = _positions(i1v, r1v, rbwv, sl, n_experts)
            pltpu.async_copy(rowsv.at[sl], xsh.at[p0], sem).wait()
            p1 = _positions(i2v, r2v, rbwv, sl, n_experts)
            pltpu.async_copy(rowsv.at[sl], xsh.at[p1], sem).wait()

    return k(x, i1, i2, r1, r2, rbw)


def _combine_call(ys, i1, i2, r1, r2, rbw, wwide, t, n_experts):
    rrows, h = ys.shape
    tpw = t // _NW
    hc = h // _L
    nch = tpw // _L

    @functools.partial(
        pl.kernel,
        out_type=jax.ShapeDtypeStruct((t, h), jnp.float32),
        mesh=_sc_mesh(),
        scratch_types=[
            pltpu.VMEM((tpw,), jnp.int32),
            pltpu.VMEM((tpw,), jnp.int32),
            pltpu.VMEM((tpw,), jnp.int32),
            pltpu.VMEM((tpw,), jnp.int32),
            pltpu.VMEM((_EPAD,), jnp.int32),
            pltpu.VMEM((tpw, _EPAD), jnp.float32),
            pltpu.VMEM((2, _L, h), jnp.float32),
            pltpu.VMEM((2, _L, h), jnp.float32),
            pltpu.SemaphoreType.DMA,
        ],
    )
    def k(ysh, i1h, i2h, r1h, r2h, rbwh, wwh, outh,
          i1v, i2v, r1v, r2v, rbwv, wv, b0, b1, sem):
        base = _wid() * tpw
        pltpu.sync_copy(i1h.at[pl.ds(base, tpw)], i1v)
        pltpu.sync_copy(i2h.at[pl.ds(base, tpw)], i2v)
        pltpu.sync_copy(r1h.at[pl.ds(base, tpw)], r1v)
        pltpu.sync_copy(r2h.at[pl.ds(base, tpw)], r2v)
        pltpu.sync_copy(rbwh, rbwv)
        pltpu.sync_copy(wwh.at[pl.ds(base, tpw)], wv)

        def start(c):
            slc = pl.ds(c * _L, _L)
            p0 = _positions(i1v, r1v, rbwv, slc, n_experts)
            p1 = _positions(i2v, r2v, rbwv, slc, n_experts)
            qb = c % 2
            d0 = pltpu.async_copy(ysh.at[p0], b0.at[qb], sem)
            d1 = pltpu.async_copy(ysh.at[p1], b1.at[qb], sem)
            return d0, d1

        descs = [None, None]
        descs[0] = start(0)
        for c in range(nch):
            if c + 1 < nch:
                descs[(c + 1) % 2] = start(c + 1)
            d0, d1 = descs[c % 2]
            d0.wait()
            d1.wait()
            qb = c % 2

            def row_body(r, _):
                m1v = wv[c * _L + r, pl.ds(0, _L)]
                m2v = wv[c * _L + r, pl.ds(_L, _L)]

                def col_body(jq, _):
                    for q in range(4):
                        sj = pl.ds((jq * 4 + q) * _L, _L)
                        b0[qb, r, sj] = (m1v * b0[qb, r, sj]
                                         + m2v * b1[qb, r, sj])
                    return 0

                lax.fori_loop(0, hc // 4, col_body, 0)
                return 0

            lax.fori_loop(0, _L, row_body, 0)
            pltpu.sync_copy(b0.at[qb], outh.at[pl.ds(base + c * _L, _L)])

    return k(ys, i1, i2, r1, r2, rbw, wwide)


def kernel(hidden_states, gate_w, w1, w2, w3):
    b, s, h = hidden_states.shape
    t = b * s
    n_experts, ffn, _ = w1.shape
    top_k = 2
    x = hidden_states.reshape(t, h)
    nl = (t * top_k) // _BM + n_experts
    rrows = nl * _BM

    gw_pad = jnp.zeros((_EPAD, h), jnp.float32).at[:n_experts].set(gate_w)
    logits_pad, ipack, wwide, countsf = _routing_call(x, gw_pad, n_experts)
    router_logits = logits_pad[:, :n_experts]

    i1 = ipack[:, 0]
    i2 = ipack[:, 1]
    r1 = ipack[:, 2]
    r2 = ipack[:, 3]

    counts = countsf[0, :n_experts].astype(jnp.int32)
    tiles_e = (counts + _BM - 1) // _BM
    start = jnp.concatenate(
        [jnp.zeros((1,), jnp.int32), jnp.cumsum(tiles_e)])
    row_base = _BM * start[:n_experts]
    rbw = jnp.repeat(row_base, _L, total_repeat_length=n_experts * _L)
    rbw = jnp.zeros((_EPAD,), jnp.int32).at[:n_experts * _L].set(rbw)
    larange = jnp.arange(nl, dtype=jnp.int32)
    gidraw = jnp.minimum(
        jnp.sum((larange[:, None] >= start[None, 1:]).astype(jnp.int32),
                axis=1), n_experts - 1).astype(jnp.int32)
    nused = start[n_experts]
    lastg = jnp.max(jnp.where(counts > 0,
                              jnp.arange(n_experts, dtype=jnp.int32), 0))
    gid = jnp.where(larange < nused, gidraw, lastg)
    nu = nused[None]
    b_l = jnp.concatenate([jnp.ones((1,), jnp.int32),
                           (gid[1:] != gid[:-1]).astype(jnp.int32)])
    b_l = b_l * (larange < nused).astype(jnp.int32)
    ordl = jnp.cumsum(b_l) - 1
    rppa = jnp.sum(b_l)[None]
    rg = jnp.zeros((n_experts,), jnp.int32).at[
        jnp.where(b_l == 1, ordl, n_experts)].set(gid, mode='drop')

    xs = _scatter_call(x, i1, i2, r1, r2, rbw, rrows, n_experts)
    ys = _gmm_call(xs, w1, w3, w2, gid, nu, ordl, rg, rppa)
    out = _combine_call(ys, i1, i2, r1, r2, rbw, wwide, t, n_experts)
    return out.reshape(b, s, h), router_logits
